# Initial kernel scaffold; baseline (speedup 1.0000x reference)
import jax
import jax.numpy as jnp
from jax.experimental import pallas as pl


def kernel(x, edge_index, bn_weight, bn_bias, w_rel0, w_root0, b0, w_rel1, w_root1, b1, w_rel2, w_root2, b2, w_rel3, w_root3, b3, lin_w, lin_b):
    raise NotImplementedError("write your pallas kernel here")


# speedup vs baseline: 2.7192x; 2.7192x over previous
import functools

import jax
import jax.numpy as jnp
from jax import lax
from jax.experimental import pallas as pl
from jax.experimental.pallas import tpu as pltpu
from jax.experimental.pallas import tpu_sc as plsc

N_NODES = 10000
N_EDGES = 320000
D = 128
N_CLASSES = 40

NC = 2
NS = 16
NW = NC * NS

NPAD = 10240
ECHUNK = 128
EPT = NPAD
E_PAD = EPT * NW
NCHUNKS = EPT // ECHUNK
STRIPE = NPAD // NS
SCHUNKS = STRIPE // ECHUNK

BLK = 512
NBLK = NPAD // BLK


def _sc_aggregate_body(h_hbm, src_hbm, dst_hbm, zrows_hbm, part_hbm,
                       sidx, didx, rows, zbuf, aggr, gsem):
  c = lax.axis_index("c")
  s = lax.axis_index("s")
  wid = c * NS + s

  pltpu.sync_copy(zrows_hbm, zbuf)
  for k in range(SCHUNKS):
    pltpu.sync_copy(zbuf, aggr.at[pl.ds(s * STRIPE + k * ECHUNK, ECHUNK)])
  plsc.subcore_barrier()

  base_e = wid * EPT

  def body(t, carry):
    base = base_e + t * ECHUNK
    pltpu.sync_copy(src_hbm.at[pl.ds(base, ECHUNK)], sidx)
    pltpu.sync_copy(dst_hbm.at[pl.ds(base, ECHUNK)], didx)
    pltpu.async_copy(h_hbm.at[sidx], rows, gsem).wait()
    pltpu.sync_copy(rows, aggr.at[didx], add=True)
    return carry

  lax.fori_loop(0, NCHUNKS, body, 0)
  plsc.subcore_barrier()

  for k in range(SCHUNKS):
    r0 = s * STRIPE + k * ECHUNK
    pltpu.sync_copy(aggr.at[pl.ds(r0, ECHUNK)], rows)
    pltpu.sync_copy(rows, part_hbm.at[pl.ds(c * NPAD + r0, ECHUNK)])


@functools.lru_cache(maxsize=None)
def _build_sc_aggregate():
  return pl.kernel(
      _sc_aggregate_body,
      out_type=jax.ShapeDtypeStruct((NC * NPAD, D), jnp.float32),
      mesh=plsc.VectorSubcoreMesh(core_axis_name="c", subcore_axis_name="s"),
      scratch_types=[
          pltpu.VMEM((ECHUNK,), jnp.int32),
          pltpu.VMEM((ECHUNK,), jnp.int32),
          pltpu.VMEM((ECHUNK, D), jnp.float32),
          pltpu.VMEM((ECHUNK, D), jnp.float32),
          pltpu.VMEM_SHARED((NPAD, D), jnp.float32),
          pltpu.SemaphoreType.DMA,
      ],
  )


def _sc_aggregate(h, src, dst, zrows):
  return _build_sc_aggregate()(h, src, dst, zrows)


def _bn_stats_body(x_ref, o_ref):
  @pl.when(pl.program_id(0) == 0)
  def _():
    o_ref[...] = jnp.zeros_like(o_ref)

  x = x_ref[...]
  o_ref[0:1, :] += jnp.sum(x, axis=0, keepdims=True)
  o_ref[1:2, :] += jnp.sum(x * x, axis=0, keepdims=True)


def _bn_apply_body(x_ref, st_ref, w_ref, b_ref, o_ref):
  st = st_ref[...]
  mean = st[0:1, :] * (1.0 / N_NODES)
  var = st[1:2, :] * (1.0 / N_NODES) - mean * mean
  inv = lax.rsqrt(var + 1e-5) * w_ref[...]
  o_ref[...] = (x_ref[...] - mean) * inv + b_ref[...]


def _batchnorm(x_pad, bn_w, bn_b):
  stats = pl.pallas_call(
      _bn_stats_body,
      out_shape=jax.ShapeDtypeStruct((2, D), jnp.float32),
      grid=(NBLK,),
      in_specs=[pl.BlockSpec((BLK, D), lambda i: (i, 0))],
      out_specs=pl.BlockSpec((2, D), lambda i: (0, 0)),
  )(x_pad)
  return pl.pallas_call(
      _bn_apply_body,
      out_shape=jax.ShapeDtypeStruct((NPAD, D), jnp.float32),
      grid=(NBLK,),
      in_specs=[
          pl.BlockSpec((BLK, D), lambda i: (i, 0)),
          pl.BlockSpec((2, D), lambda i: (0, 0)),
          pl.BlockSpec((1, D), lambda i: (0, 0)),
          pl.BlockSpec((1, D), lambda i: (0, 0)),
      ],
      out_specs=pl.BlockSpec((BLK, D), lambda i: (i, 0)),
  )(x_pad, stats, bn_w.reshape(1, D), bn_b.reshape(1, D))


def _combine_body(relu, p0_ref, p1_ref, h_ref, wr_ref, ww_ref, b_ref, o_ref):
  aggr = p0_ref[...] + p1_ref[...]
  acc = jnp.dot(aggr, wr_ref[...], preferred_element_type=jnp.float32)
  acc += jnp.dot(h_ref[...], ww_ref[...], preferred_element_type=jnp.float32)
  acc += b_ref[...]
  if relu:
    acc = jnp.maximum(acc, 0.0)
  o_ref[...] = acc


def _combine(part, h, w_rel, w_root, b, relu):
  return pl.pallas_call(
      functools.partial(_combine_body, relu),
      out_shape=jax.ShapeDtypeStruct((NPAD, D), jnp.float32),
      grid=(NBLK,),
      in_specs=[
          pl.BlockSpec((BLK, D), lambda i: (i, 0)),
          pl.BlockSpec((BLK, D), lambda i: (i, 0)),
          pl.BlockSpec((BLK, D), lambda i: (i, 0)),
          pl.BlockSpec((D, D), lambda i: (0, 0)),
          pl.BlockSpec((D, D), lambda i: (0, 0)),
          pl.BlockSpec((1, D), lambda i: (0, 0)),
      ],
      out_specs=pl.BlockSpec((BLK, D), lambda i: (i, 0)),
  )(part[:NPAD], part[NPAD:], h, w_rel, w_root, b.reshape(1, D))


def _head_body(h_ref, lw_ref, lb_ref, o_ref, s_ref):
  i = pl.program_id(0)

  @pl.when(i == 0)
  def _():
    s_ref[...] = jnp.zeros_like(s_ref)

  rows = i * BLK + lax.broadcasted_iota(jnp.int32, (BLK, 1), 0)
  x = jnp.where(rows < N_NODES, h_ref[...], 0.0)
  s_ref[...] += jnp.sum(x, axis=0, keepdims=True)

  @pl.when(i == NBLK - 1)
  def _():
    pooled = s_ref[...] * (1.0 / N_NODES)
    o_ref[...] = (
        jnp.dot(pooled, lw_ref[...], preferred_element_type=jnp.float32)
        + lb_ref[...]
    )


def _head(h, lin_w, lin_b):
  return pl.pallas_call(
      _head_body,
      out_shape=jax.ShapeDtypeStruct((1, N_CLASSES), jnp.float32),
      grid=(NBLK,),
      in_specs=[
          pl.BlockSpec((BLK, D), lambda i: (i, 0)),
          pl.BlockSpec((D, N_CLASSES), lambda i: (0, 0)),
          pl.BlockSpec((1, N_CLASSES), lambda i: (0, 0)),
      ],
      out_specs=pl.BlockSpec((1, N_CLASSES), lambda i: (0, 0)),
      scratch_shapes=[pltpu.VMEM((1, D), jnp.float32)],
  )(h, lin_w, lin_b.reshape(1, N_CLASSES))


def kernel(x, edge_index, bn_weight, bn_bias,
           w_rel0, w_root0, b0, w_rel1, w_root1, b1,
           w_rel2, w_root2, b2, w_rel3, w_root3, b3,
           lin_w, lin_b):
  x_pad = jnp.zeros((NPAD, D), jnp.float32).at[:N_NODES].set(x)

  src = edge_index[0].astype(jnp.int32)
  dst = edge_index[1].astype(jnp.int32)
  npadedge = E_PAD - N_EDGES
  src = jnp.concatenate([src, jnp.zeros((npadedge,), jnp.int32)])
  dst = jnp.concatenate([dst, jnp.full((npadedge,), N_NODES, jnp.int32)])
  zrows = jnp.zeros((ECHUNK, D), jnp.float32)

  h = _batchnorm(x_pad, bn_weight, bn_bias)
  layers = [(w_rel0, w_root0, b0, True), (w_rel1, w_root1, b1, True),
            (w_rel2, w_root2, b2, True), (w_rel3, w_root3, b3, False)]
  for w_rel, w_root, b, relu in layers:
    part = _sc_aggregate(h, src, dst, zrows)
    h = _combine(part, h, w_rel, w_root, b, relu)

  return _head(h, lin_w, lin_b)


# speedup vs baseline: 3.3490x; 1.2316x over previous
import functools

import jax
import jax.numpy as jnp
from jax import lax
from jax.experimental import pallas as pl
from jax.experimental.pallas import tpu as pltpu
from jax.experimental.pallas import tpu_sc as plsc

N_NODES = 10000
N_EDGES = 320000
D = 128
N_CLASSES = 40

NC = 2
NS = 16
NW = NC * NS

NPAD = 10240
ECHUNK = 128
EPT = NPAD
E_PAD = EPT * NW
NCHUNKS = EPT // ECHUNK
STRIPE = NPAD // NS
SCHUNKS = STRIPE // ECHUNK

BLK = 512
NBLK = NPAD // BLK


HCHUNKS = NCHUNKS // 2


def _sc_aggregate_body(h_hbm, srcb_hbm, dstb_hbm, zrows_hbm, part_hbm,
                       sidx, didx, rows0, rows1, aggr, gsem0, gsem1):
  c = lax.axis_index("c")
  s = lax.axis_index("s")
  wid = c * NS + s

  pltpu.sync_copy(zrows_hbm, rows0)
  for k in range(SCHUNKS):
    pltpu.sync_copy(rows0, aggr.at[pl.ds(s * STRIPE + k * ECHUNK, ECHUNK)])
  plsc.subcore_barrier()

  def gather(t, buf, sem):
    pltpu.async_copy(h_hbm.at[sidx.at[t]], buf, sem)

  def drain(buf, sem):
    pltpu.make_async_copy(h_hbm.at[pl.ds(0, ECHUNK)], buf, sem).wait()

  def scatter(t, buf):
    pltpu.sync_copy(buf, aggr.at[didx.at[t]], add=True)

  for half in range(2):
    base_row = wid * NCHUNKS + half * HCHUNKS
    pltpu.sync_copy(srcb_hbm.at[pl.ds(base_row, HCHUNKS)], sidx)
    pltpu.sync_copy(dstb_hbm.at[pl.ds(base_row, HCHUNKS)], didx)

    gather(0, rows0, gsem0)

    def body(i, carry):
      a = 2 * i
      gather(a + 1, rows1, gsem1)
      drain(rows0, gsem0)
      scatter(a, rows0)
      gather(a + 2, rows0, gsem0)
      drain(rows1, gsem1)
      scatter(a + 1, rows1)
      return carry

    lax.fori_loop(0, HCHUNKS // 2 - 1, body, 0)
    gather(HCHUNKS - 1, rows1, gsem1)
    drain(rows0, gsem0)
    scatter(HCHUNKS - 2, rows0)
    drain(rows1, gsem1)
    scatter(HCHUNKS - 1, rows1)

  plsc.subcore_barrier()

  for k in range(SCHUNKS):
    r0 = s * STRIPE + k * ECHUNK
    pltpu.sync_copy(aggr.at[pl.ds(r0, ECHUNK)], rows0)
    pltpu.sync_copy(rows0, part_hbm.at[pl.ds(c * NPAD + r0, ECHUNK)])


@functools.lru_cache(maxsize=None)
def _build_sc_aggregate():
  return pl.kernel(
      _sc_aggregate_body,
      out_type=jax.ShapeDtypeStruct((NC * NPAD, D), jnp.float32),
      mesh=plsc.VectorSubcoreMesh(core_axis_name="c", subcore_axis_name="s"),
      scratch_types=[
          pltpu.VMEM((NCHUNKS // 2, ECHUNK), jnp.int32),
          pltpu.VMEM((NCHUNKS // 2, ECHUNK), jnp.int32),
          pltpu.VMEM((ECHUNK, D), jnp.float32),
          pltpu.VMEM((ECHUNK, D), jnp.float32),
          pltpu.VMEM_SHARED((NPAD, D), jnp.float32),
          pltpu.SemaphoreType.DMA,
          pltpu.SemaphoreType.DMA,
      ],
  )


def _sc_aggregate(h, src, dst, zrows):
  return _build_sc_aggregate()(h, src, dst, zrows)


def _bn_stats_body(x_ref, o_ref):
  @pl.when(pl.program_id(0) == 0)
  def _():
    o_ref[...] = jnp.zeros_like(o_ref)

  x = x_ref[...]
  o_ref[0:1, :] += jnp.sum(x, axis=0, keepdims=True)
  o_ref[1:2, :] += jnp.sum(x * x, axis=0, keepdims=True)


def _bn_apply_body(x_ref, st_ref, w_ref, b_ref, o_ref):
  st = st_ref[...]
  mean = st[0:1, :] * (1.0 / N_NODES)
  var = st[1:2, :] * (1.0 / N_NODES) - mean * mean
  inv = lax.rsqrt(var + 1e-5) * w_ref[...]
  o_ref[...] = (x_ref[...] - mean) * inv + b_ref[...]


def _batchnorm(x_pad, bn_w, bn_b):
  stats = pl.pallas_call(
      _bn_stats_body,
      out_shape=jax.ShapeDtypeStruct((2, D), jnp.float32),
      grid=(NBLK,),
      in_specs=[pl.BlockSpec((BLK, D), lambda i: (i, 0))],
      out_specs=pl.BlockSpec((2, D), lambda i: (0, 0)),
  )(x_pad)
  return pl.pallas_call(
      _bn_apply_body,
      out_shape=jax.ShapeDtypeStruct((NPAD, D), jnp.float32),
      grid=(NBLK,),
      in_specs=[
          pl.BlockSpec((BLK, D), lambda i: (i, 0)),
          pl.BlockSpec((2, D), lambda i: (0, 0)),
          pl.BlockSpec((1, D), lambda i: (0, 0)),
          pl.BlockSpec((1, D), lambda i: (0, 0)),
      ],
      out_specs=pl.BlockSpec((BLK, D), lambda i: (i, 0)),
  )(x_pad, stats, bn_w.reshape(1, D), bn_b.reshape(1, D))


def _combine_body(relu, p0_ref, p1_ref, h_ref, wr_ref, ww_ref, b_ref, o_ref):
  aggr = p0_ref[...] + p1_ref[...]
  acc = jnp.dot(aggr, wr_ref[...], preferred_element_type=jnp.float32)
  acc += jnp.dot(h_ref[...], ww_ref[...], preferred_element_type=jnp.float32)
  acc += b_ref[...]
  if relu:
    acc = jnp.maximum(acc, 0.0)
  o_ref[...] = acc


def _combine(part, h, w_rel, w_root, b, relu):
  return pl.pallas_call(
      functools.partial(_combine_body, relu),
      out_shape=jax.ShapeDtypeStruct((NPAD, D), jnp.float32),
      grid=(NBLK,),
      in_specs=[
          pl.BlockSpec((BLK, D), lambda i: (i, 0)),
          pl.BlockSpec((BLK, D), lambda i: (i, 0)),
          pl.BlockSpec((BLK, D), lambda i: (i, 0)),
          pl.BlockSpec((D, D), lambda i: (0, 0)),
          pl.BlockSpec((D, D), lambda i: (0, 0)),
          pl.BlockSpec((1, D), lambda i: (0, 0)),
      ],
      out_specs=pl.BlockSpec((BLK, D), lambda i: (i, 0)),
  )(part[:NPAD], part[NPAD:], h, w_rel, w_root, b.reshape(1, D))


def _head_body(h_ref, lw_ref, lb_ref, o_ref, s_ref):
  i = pl.program_id(0)

  @pl.when(i == 0)
  def _():
    s_ref[...] = jnp.zeros_like(s_ref)

  rows = i * BLK + lax.broadcasted_iota(jnp.int32, (BLK, 1), 0)
  x = jnp.where(rows < N_NODES, h_ref[...], 0.0)
  s_ref[...] += jnp.sum(x, axis=0, keepdims=True)

  @pl.when(i == NBLK - 1)
  def _():
    pooled = s_ref[...] * (1.0 / N_NODES)
    o_ref[...] = (
        jnp.dot(pooled, lw_ref[...], preferred_element_type=jnp.float32)
        + lb_ref[...]
    )


def _head(h, lin_w, lin_b):
  return pl.pallas_call(
      _head_body,
      out_shape=jax.ShapeDtypeStruct((1, N_CLASSES), jnp.float32),
      grid=(NBLK,),
      in_specs=[
          pl.BlockSpec((BLK, D), lambda i: (i, 0)),
          pl.BlockSpec((D, N_CLASSES), lambda i: (0, 0)),
          pl.BlockSpec((1, N_CLASSES), lambda i: (0, 0)),
      ],
      out_specs=pl.BlockSpec((1, N_CLASSES), lambda i: (0, 0)),
      scratch_shapes=[pltpu.VMEM((1, D), jnp.float32)],
  )(h, lin_w, lin_b.reshape(1, N_CLASSES))


def kernel(x, edge_index, bn_weight, bn_bias,
           w_rel0, w_root0, b0, w_rel1, w_root1, b1,
           w_rel2, w_root2, b2, w_rel3, w_root3, b3,
           lin_w, lin_b):
  x_pad = jnp.zeros((NPAD, D), jnp.float32).at[:N_NODES].set(x)

  src = edge_index[0].astype(jnp.int32)
  dst = edge_index[1].astype(jnp.int32)
  npadedge = E_PAD - N_EDGES
  pad_dst = N_NODES + jnp.arange(npadedge, dtype=jnp.int32) % (NPAD - N_NODES)
  src = jnp.concatenate([src, jnp.zeros((npadedge,), jnp.int32)])
  dst = jnp.concatenate([dst, pad_dst])
  src = src.reshape(E_PAD // ECHUNK, ECHUNK)
  dst = dst.reshape(E_PAD // ECHUNK, ECHUNK)
  zrows = jnp.zeros((ECHUNK, D), jnp.float32)

  h = _batchnorm(x_pad, bn_weight, bn_bias)
  layers = [(w_rel0, w_root0, b0, True), (w_rel1, w_root1, b1, True),
            (w_rel2, w_root2, b2, True), (w_rel3, w_root3, b3, False)]
  for w_rel, w_root, b, relu in layers:
    part = _sc_aggregate(h, src, dst, zrows)
    h = _combine(part, h, w_rel, w_root, b, relu)

  return _head(h, lin_w, lin_b)


# speedup vs baseline: 3.3497x; 1.0002x over previous
import functools

import jax
import jax.numpy as jnp
from jax import lax
from jax.experimental import pallas as pl
from jax.experimental.pallas import tpu as pltpu
from jax.experimental.pallas import tpu_sc as plsc

N_NODES = 10000
N_EDGES = 320000
D = 128
N_CLASSES = 40

NC = 2
NS = 16
NW = NC * NS

NPAD = 10240
ECHUNK = 128
EPT = NPAD
E_PAD = EPT * NW
NCHUNKS = EPT // ECHUNK
HCHUNKS = NCHUNKS // 2
STRIPE = NPAD // NS
SCHUNKS = STRIPE // ECHUNK

BLK = 512
NBLK = NPAD // BLK


def _sc_aggregate_body(h_hbm, srcb_hbm, dstb_hbm, zrows_hbm, part_hbm,
                       sidx, didx, rows0, rows1, aggr, gsem0, gsem1):
  c = lax.axis_index("c")
  s = lax.axis_index("s")
  wid = c * NS + s

  pltpu.sync_copy(zrows_hbm, rows0)
  for k in range(SCHUNKS):
    pltpu.sync_copy(rows0, aggr.at[pl.ds(s * STRIPE + k * ECHUNK, ECHUNK)])
  plsc.subcore_barrier()

  def gather(t, buf, sem):
    pltpu.async_copy(h_hbm.at[sidx.at[t]], buf, sem)

  def drain(buf, sem):
    pltpu.make_async_copy(h_hbm.at[pl.ds(0, ECHUNK)], buf, sem).wait()

  def scatter(t, buf):
    pltpu.sync_copy(buf, aggr.at[didx.at[t]], add=True)

  for half in range(2):
    base_row = wid * NCHUNKS + half * HCHUNKS
    pltpu.sync_copy(srcb_hbm.at[pl.ds(base_row, HCHUNKS)], sidx)
    pltpu.sync_copy(dstb_hbm.at[pl.ds(base_row, HCHUNKS)], didx)

    gather(0, rows0, gsem0)

    def body(i, carry):
      a = 2 * i
      gather(a + 1, rows1, gsem1)
      drain(rows0, gsem0)
      scatter(a, rows0)
      gather(a + 2, rows0, gsem0)
      drain(rows1, gsem1)
      scatter(a + 1, rows1)
      return carry

    lax.fori_loop(0, HCHUNKS // 2 - 1, body, 0)
    gather(HCHUNKS - 1, rows1, gsem1)
    drain(rows0, gsem0)
    scatter(HCHUNKS - 2, rows0)
    drain(rows1, gsem1)
    scatter(HCHUNKS - 1, rows1)

  plsc.subcore_barrier()

  for k in range(SCHUNKS):
    r0 = s * STRIPE + k * ECHUNK
    pltpu.sync_copy(aggr.at[pl.ds(r0, ECHUNK)], rows0)
    pltpu.sync_copy(rows0, part_hbm.at[pl.ds(c * NPAD + r0, ECHUNK)])


@functools.lru_cache(maxsize=None)
def _build_sc_aggregate():
  return pl.kernel(
      _sc_aggregate_body,
      out_type=jax.ShapeDtypeStruct((NC * NPAD, D), jnp.float32),
      mesh=plsc.VectorSubcoreMesh(core_axis_name="c", subcore_axis_name="s"),
      scratch_types=[
          pltpu.VMEM((HCHUNKS, ECHUNK), jnp.int32),
          pltpu.VMEM((HCHUNKS, ECHUNK), jnp.int32),
          pltpu.VMEM((ECHUNK, D), jnp.float32),
          pltpu.VMEM((ECHUNK, D), jnp.float32),
          pltpu.VMEM_SHARED((NPAD, D), jnp.float32),
          pltpu.SemaphoreType.DMA,
          pltpu.SemaphoreType.DMA,
      ],
  )


def _sc_aggregate(h, src, dst, zrows):
  return _build_sc_aggregate()(h, src, dst, zrows)


def _bn_stats_body(x_ref, o_ref):
  @pl.when(pl.program_id(0) == 0)
  def _():
    o_ref[...] = jnp.zeros_like(o_ref)

  x = x_ref[...]
  o_ref[0:1, :] += jnp.sum(x, axis=0, keepdims=True)
  o_ref[1:2, :] += jnp.sum(x * x, axis=0, keepdims=True)


def _bn_apply_body(x_ref, st_ref, w_ref, b_ref, o_ref):
  st = st_ref[...]
  mean = st[0:1, :] * (1.0 / N_NODES)
  var = st[1:2, :] * (1.0 / N_NODES) - mean * mean
  inv = lax.rsqrt(var + 1e-5) * w_ref[...]
  o_ref[...] = (x_ref[...] - mean) * inv + b_ref[...]


def _batchnorm(x_pad, bn_w, bn_b):
  stats = pl.pallas_call(
      _bn_stats_body,
      out_shape=jax.ShapeDtypeStruct((2, D), jnp.float32),
      grid=(NBLK,),
      in_specs=[pl.BlockSpec((BLK, D), lambda i: (i, 0))],
      out_specs=pl.BlockSpec((2, D), lambda i: (0, 0)),
  )(x_pad)
  return pl.pallas_call(
      _bn_apply_body,
      out_shape=jax.ShapeDtypeStruct((NPAD, D), jnp.float32),
      grid=(NBLK,),
      in_specs=[
          pl.BlockSpec((BLK, D), lambda i: (i, 0)),
          pl.BlockSpec((2, D), lambda i: (0, 0)),
          pl.BlockSpec((1, D), lambda i: (0, 0)),
          pl.BlockSpec((1, D), lambda i: (0, 0)),
      ],
      out_specs=pl.BlockSpec((BLK, D), lambda i: (i, 0)),
  )(x_pad, stats, bn_w.reshape(1, D), bn_b.reshape(1, D))


def _combine_body(relu, p0_ref, p1_ref, h_ref, wr_ref, ww_ref, b_ref, o_ref):
  aggr = p0_ref[...] + p1_ref[...]
  acc = jnp.dot(aggr, wr_ref[...], preferred_element_type=jnp.float32)
  acc += jnp.dot(h_ref[...], ww_ref[...], preferred_element_type=jnp.float32)
  acc += b_ref[...]
  if relu:
    acc = jnp.maximum(acc, 0.0)
  o_ref[...] = acc


def _combine(part, h, w_rel, w_root, b, relu):
  return pl.pallas_call(
      functools.partial(_combine_body, relu),
      out_shape=jax.ShapeDtypeStruct((NPAD, D), jnp.float32),
      grid=(NBLK,),
      in_specs=[
          pl.BlockSpec((BLK, D), lambda i: (i, 0)),
          pl.BlockSpec((BLK, D), lambda i: (i, 0)),
          pl.BlockSpec((BLK, D), lambda i: (i, 0)),
          pl.BlockSpec((D, D), lambda i: (0, 0)),
          pl.BlockSpec((D, D), lambda i: (0, 0)),
          pl.BlockSpec((1, D), lambda i: (0, 0)),
      ],
      out_specs=pl.BlockSpec((BLK, D), lambda i: (i, 0)),
  )(part[:NPAD], part[NPAD:], h, w_rel, w_root, b.reshape(1, D))


def _head_body(h_ref, lw_ref, lb_ref, o_ref, s_ref):
  i = pl.program_id(0)

  @pl.when(i == 0)
  def _():
    s_ref[...] = jnp.zeros_like(s_ref)

  rows = i * BLK + lax.broadcasted_iota(jnp.int32, (BLK, 1), 0)
  x = jnp.where(rows < N_NODES, h_ref[...], 0.0)
  s_ref[...] += jnp.sum(x, axis=0, keepdims=True)

  @pl.when(i == NBLK - 1)
  def _():
    pooled = s_ref[...] * (1.0 / N_NODES)
    o_ref[...] = (
        jnp.dot(pooled, lw_ref[...], preferred_element_type=jnp.float32)
        + lb_ref[...]
    )


def _head(h, lin_w, lin_b):
  return pl.pallas_call(
      _head_body,
      out_shape=jax.ShapeDtypeStruct((1, N_CLASSES), jnp.float32),
      grid=(NBLK,),
      in_specs=[
          pl.BlockSpec((BLK, D), lambda i: (i, 0)),
          pl.BlockSpec((D, N_CLASSES), lambda i: (0, 0)),
          pl.BlockSpec((1, N_CLASSES), lambda i: (0, 0)),
      ],
      out_specs=pl.BlockSpec((1, N_CLASSES), lambda i: (0, 0)),
      scratch_shapes=[pltpu.VMEM((1, D), jnp.float32)],
  )(h, lin_w, lin_b.reshape(1, N_CLASSES))


def kernel(x, edge_index, bn_weight, bn_bias,
           w_rel0, w_root0, b0, w_rel1, w_root1, b1,
           w_rel2, w_root2, b2, w_rel3, w_root3, b3,
           lin_w, lin_b):
  x_pad = jnp.zeros((NPAD, D), jnp.float32).at[:N_NODES].set(x)

  src = edge_index[0].astype(jnp.int32)
  dst = edge_index[1].astype(jnp.int32)
  npadedge = E_PAD - N_EDGES
  pad_dst = N_NODES + jnp.arange(npadedge, dtype=jnp.int32) % (NPAD - N_NODES)
  src = jnp.concatenate([src, jnp.zeros((npadedge,), jnp.int32)])
  dst = jnp.concatenate([dst, pad_dst])
  src = src.reshape(E_PAD // ECHUNK, ECHUNK)
  dst = dst.reshape(E_PAD // ECHUNK, ECHUNK)
  zrows = jnp.zeros((ECHUNK, D), jnp.float32)

  h = _batchnorm(x_pad, bn_weight, bn_bias)
  layers = [(w_rel0, w_root0, b0, True), (w_rel1, w_root1, b1, True),
            (w_rel2, w_root2, b2, True), (w_rel3, w_root3, b3, False)]
  for w_rel, w_root, b, relu in layers:
    part = _sc_aggregate(h, src, dst, zrows)
    h = _combine(part, h, w_rel, w_root, b, relu)

  return _head(h, lin_w, lin_b)
